# Initial kernel scaffold; baseline (speedup 1.0000x reference)
import jax
import jax.numpy as jnp
from jax.experimental import pallas as pl


def kernel(x, edge_index_all, batch, W1, att_l1, att_r1, b1, gn_a1, gn_g1, gn_b1, W2, att_l2, att_r2, b2, gn_a2, gn_g2, gn_b2):
    raise NotImplementedError("write your pallas kernel here")


# speedup vs baseline: 1.0136x; 1.0136x over previous
import functools

import jax
import jax.numpy as jnp
from jax.experimental import pallas as pl

_NUM_GRAPHS = 64


def _mm_att_body(x_ref, w_ref, a_ref, h_ref, alr_ref):
    x = x_ref[...]
    h = jnp.dot(x, w_ref[...], preferred_element_type=jnp.float32)
    h_ref[...] = h
    alr_ref[...] = jnp.dot(h, a_ref[...], preferred_element_type=jnp.float32)


@functools.partial(jax.jit, static_argnames=("bn",))
def _mm_att(x, w, amat, bn=512):
    n, k = x.shape
    ho = w.shape[1]
    na = amat.shape[1]
    grid = (n // bn,)
    h, alr = pl.pallas_call(
        _mm_att_body,
        grid=grid,
        in_specs=[
            pl.BlockSpec((bn, k), lambda i: (i, 0)),
            pl.BlockSpec((k, ho), lambda i: (0, 0)),
            pl.BlockSpec((ho, na), lambda i: (0, 0)),
        ],
        out_specs=[
            pl.BlockSpec((bn, ho), lambda i: (i, 0)),
            pl.BlockSpec((bn, na), lambda i: (i, 0)),
        ],
        out_shape=[
            jax.ShapeDtypeStruct((n, ho), jnp.float32),
            jax.ShapeDtypeStruct((n, na), jnp.float32),
        ],
    )(x, w, amat)
    return h, alr


def _gn_body(y_ref, b_ref, a_ref, g_ref, be_ref, out_ref, emb_ref):
    b = b_ref[...]
    npad = b.shape[1]
    gids = jax.lax.broadcasted_iota(jnp.int32, (_NUM_GRAPHS, npad), 0)
    onehot = (b == gids).astype(jnp.float32)
    cnt = jnp.maximum(jnp.sum(onehot, axis=1, keepdims=True), 1.0)
    y = y_ref[...]
    mean = jnp.dot(onehot, y, preferred_element_type=jnp.float32) / cnt
    sub = y - a_ref[...] * jnp.dot(onehot.T, mean,
                                   preferred_element_type=jnp.float32)
    var = jnp.dot(onehot, sub * sub, preferred_element_type=jnp.float32) / cnt
    varb = jnp.dot(onehot.T, var, preferred_element_type=jnp.float32)
    out = g_ref[...] * sub * jax.lax.rsqrt(varb + 1e-5) + be_ref[...]
    out = jnp.maximum(out, 0.0)
    out_ref[...] = out
    emb_ref[...] = jnp.dot(onehot, out, preferred_element_type=jnp.float32)


@functools.partial(jax.jit, static_argnames=("bc",))
def _gn(y, batch_row, a, g, be, bc=256):
    npad, c = y.shape
    grid = (c // bc,)
    out, emb = pl.pallas_call(
        _gn_body,
        grid=grid,
        in_specs=[
            pl.BlockSpec((npad, bc), lambda i: (0, i)),
            pl.BlockSpec((1, npad), lambda i: (0, 0)),
            pl.BlockSpec((1, bc), lambda i: (0, i)),
            pl.BlockSpec((1, bc), lambda i: (0, i)),
            pl.BlockSpec((1, bc), lambda i: (0, i)),
        ],
        out_specs=[
            pl.BlockSpec((npad, bc), lambda i: (0, i)),
            pl.BlockSpec((_NUM_GRAPHS, bc), lambda i: (0, i)),
        ],
        out_shape=[
            jax.ShapeDtypeStruct((npad, c), jnp.float32),
            jax.ShapeDtypeStruct((_NUM_GRAPHS, c), jnp.float32),
        ],
    )(y, batch_row, a.reshape(1, -1), g.reshape(1, -1), be.reshape(1, -1))
    return out, emb


def _att_mat(att_l, att_r):
    heads, hid = att_l.shape
    eye = jnp.eye(heads, dtype=jnp.float32)
    al = (att_l[:, :, None] * eye[:, None, :]).reshape(heads * hid, heads)
    ar = (att_r[:, :, None] * eye[:, None, :]).reshape(heads * hid, heads)
    return jnp.concatenate([al, ar], axis=1)


def _edge_softmax_agg(h, alr, src, dst, heads, hid, npad):
    al = alr[:, :heads]
    ar = alr[:, heads:2 * heads]
    e = jax.nn.leaky_relu(al[src] + ar[dst], negative_slope=0.2)
    m = jax.ops.segment_max(e, dst, num_segments=npad)
    m = jnp.where(jnp.isfinite(m), m, 0.0)
    ex = jnp.exp(e - m[dst])
    s = jax.ops.segment_sum(ex, dst, num_segments=npad)
    attn = ex / (s[dst] + 1e-16)
    hh = h.reshape(npad, heads, hid)
    msg = hh[src] * attn[:, :, None]
    return jax.ops.segment_sum(msg, dst, num_segments=npad)


def kernel(x, edge_index_all, batch, W1, att_l1, att_r1, b1, gn_a1, gn_g1,
           gn_b1, W2, att_l2, att_r2, b2, gn_a2, gn_g2, gn_b2):
    n = x.shape[0]
    heads, hid = att_l1.shape
    npad = ((n + 511) // 512) * 512

    xp = jnp.pad(x, ((0, npad - n), (0, 0)))
    batch_row = jnp.pad(batch, (0, npad - n),
                        constant_values=-1).reshape(1, npad).astype(jnp.int32)
    src = edge_index_all[0]
    dst = edge_index_all[1]

    h1, alr1 = _mm_att(xp, W1, _att_mat(att_l1, att_r1))
    agg1 = _edge_softmax_agg(h1, alr1, src, dst, heads, hid, npad)
    y1 = agg1.reshape(npad, heads * hid) + b1
    out1, _ = _gn(y1, batch_row, gn_a1, gn_g1, gn_b1)

    h2, alr2 = _mm_att(out1, W2, _att_mat(att_l2, att_r2))
    agg2 = _edge_softmax_agg(h2, alr2, src, dst, heads, hid, npad)
    y2 = agg2.mean(axis=1) + b2
    _, emb = _gn(y2, batch_row, gn_a2, gn_g2, gn_b2)
    return emb


# speedup vs baseline: 5.0138x; 4.9465x over previous
import functools

import jax
import jax.numpy as jnp
from jax.experimental import pallas as pl

_NUM_GRAPHS = 64


def _mm_att_body(x_ref, w_ref, a_ref, h_ref, alr_ref):
    x = x_ref[...]
    h = jnp.dot(x, w_ref[...], preferred_element_type=jnp.float32)
    h_ref[...] = h
    alr_ref[...] = jnp.dot(h, a_ref[...], preferred_element_type=jnp.float32)


@functools.partial(jax.jit, static_argnames=("bn",))
def _mm_att(x, w, amat, bn=512):
    n, k = x.shape
    ho = w.shape[1]
    na = amat.shape[1]
    grid = (n // bn,)
    h, alr = pl.pallas_call(
        _mm_att_body,
        grid=grid,
        in_specs=[
            pl.BlockSpec((bn, k), lambda i: (i, 0)),
            pl.BlockSpec((k, ho), lambda i: (0, 0)),
            pl.BlockSpec((ho, na), lambda i: (0, 0)),
        ],
        out_specs=[
            pl.BlockSpec((bn, ho), lambda i: (i, 0)),
            pl.BlockSpec((bn, na), lambda i: (i, 0)),
        ],
        out_shape=[
            jax.ShapeDtypeStruct((n, ho), jnp.float32),
            jax.ShapeDtypeStruct((n, na), jnp.float32),
        ],
    )(x, w, amat)
    return h, alr


def _gn_body(y_ref, b_ref, a_ref, g_ref, be_ref, out_ref, emb_ref):
    b = b_ref[...]
    npad = b.shape[1]
    gids = jax.lax.broadcasted_iota(jnp.int32, (_NUM_GRAPHS, npad), 0)
    onehot = (b == gids).astype(jnp.float32)
    cnt = jnp.maximum(jnp.sum(onehot, axis=1, keepdims=True), 1.0)
    y = y_ref[...]
    mean = jnp.dot(onehot, y, preferred_element_type=jnp.float32) / cnt
    sub = y - a_ref[...] * jnp.dot(onehot.T, mean,
                                   preferred_element_type=jnp.float32)
    var = jnp.dot(onehot, sub * sub, preferred_element_type=jnp.float32) / cnt
    varb = jnp.dot(onehot.T, var, preferred_element_type=jnp.float32)
    out = g_ref[...] * sub * jax.lax.rsqrt(varb + 1e-5) + be_ref[...]
    out = jnp.maximum(out, 0.0)
    out_ref[...] = out
    emb_ref[...] = jnp.dot(onehot, out, preferred_element_type=jnp.float32)


@functools.partial(jax.jit, static_argnames=("bc",))
def _gn(y, batch_row, a, g, be, bc=256):
    npad, c = y.shape
    grid = (c // bc,)
    out, emb = pl.pallas_call(
        _gn_body,
        grid=grid,
        in_specs=[
            pl.BlockSpec((npad, bc), lambda i: (0, i)),
            pl.BlockSpec((1, npad), lambda i: (0, 0)),
            pl.BlockSpec((1, bc), lambda i: (0, i)),
            pl.BlockSpec((1, bc), lambda i: (0, i)),
            pl.BlockSpec((1, bc), lambda i: (0, i)),
        ],
        out_specs=[
            pl.BlockSpec((npad, bc), lambda i: (0, i)),
            pl.BlockSpec((_NUM_GRAPHS, bc), lambda i: (0, i)),
        ],
        out_shape=[
            jax.ShapeDtypeStruct((npad, c), jnp.float32),
            jax.ShapeDtypeStruct((_NUM_GRAPHS, c), jnp.float32),
        ],
    )(y, batch_row, a.reshape(1, -1), g.reshape(1, -1), be.reshape(1, -1))
    return out, emb


def _att_mat(att_l, att_r):
    heads, hid = att_l.shape
    eye = jnp.eye(heads, dtype=jnp.float32)
    al = (att_l[:, :, None] * eye[:, None, :]).reshape(heads * hid, heads)
    ar = (att_r[:, :, None] * eye[:, None, :]).reshape(heads * hid, heads)
    return jnp.concatenate([al, ar], axis=1)


def _edge_softmax_agg(h, alr, src, dst, heads, hid, npad, mean_heads):
    al = alr[:, :heads]
    ar = alr[:, heads:2 * heads]
    e = jax.nn.leaky_relu(al[src] + ar[dst], negative_slope=0.2)
    m = jax.ops.segment_max(e, dst, num_segments=npad,
                            indices_are_sorted=True)
    m = jnp.where(jnp.isfinite(m), m, 0.0)
    ex = jnp.exp(e - m[dst])
    s = jax.ops.segment_sum(ex, dst, num_segments=npad,
                            indices_are_sorted=True)
    attn = ex / (s[dst] + 1e-16)
    hh = h.reshape(npad, heads, hid)
    if mean_heads:
        msg = jnp.einsum('ehf,eh->ef', hh[src], attn) / heads
    else:
        msg = (hh[src] * attn[:, :, None]).reshape(-1, heads * hid)
    return jax.ops.segment_sum(msg, dst, num_segments=npad,
                               indices_are_sorted=True)


def kernel(x, edge_index_all, batch, W1, att_l1, att_r1, b1, gn_a1, gn_g1,
           gn_b1, W2, att_l2, att_r2, b2, gn_a2, gn_g2, gn_b2):
    n = x.shape[0]
    heads, hid = att_l1.shape
    npad = ((n + 511) // 512) * 512

    xp = jnp.pad(x, ((0, npad - n), (0, 0)))
    batch_row = jnp.pad(batch, (0, npad - n),
                        constant_values=-1).reshape(1, npad).astype(jnp.int32)
    perm = jnp.argsort(edge_index_all[1])
    src = edge_index_all[0][perm]
    dst = edge_index_all[1][perm]

    h1, alr1 = _mm_att(xp, W1, _att_mat(att_l1, att_r1))
    agg1 = _edge_softmax_agg(h1, alr1, src, dst, heads, hid, npad, False)
    y1 = agg1 + b1
    out1, _ = _gn(y1, batch_row, gn_a1, gn_g1, gn_b1)

    h2, alr2 = _mm_att(out1, W2, _att_mat(att_l2, att_r2))
    agg2 = _edge_softmax_agg(h2, alr2, src, dst, heads, hid, npad, True)
    y2 = agg2 + b2
    _, emb = _gn(y2, batch_row, gn_a2, gn_g2, gn_b2)
    return emb
